# baseline (speedup 1.0000x reference)
import functools
import jax
import jax.numpy as jnp
from jax.experimental import pallas as pl
from jax.experimental.pallas import tpu as pltpu

B, L, D = 16384, 200, 16
BB = 64


def _tc_body(x_ref, m_ref, o_ref):
    x = x_ref[...]
    m = m_ref[...]
    o_ref[...] = jnp.sum(x * m, axis=1)


@jax.jit
def _run(inputs, maskf):
    return pl.pallas_call(
        _tc_body,
        grid=(B // BB,),
        in_specs=[
            pl.BlockSpec((BB, L, D), lambda i: (i, 0, 0)),
            pl.BlockSpec((BB, L, 1), lambda i: (i, 0, 0)),
        ],
        out_specs=pl.BlockSpec((BB, D), lambda i: (i, 0)),
        out_shape=jax.ShapeDtypeStruct((B, D), jnp.float32),
        compiler_params=pltpu.CompilerParams(
            dimension_semantics=("arbitrary",)),
    )(inputs, maskf)


def kernel(inputs, mask):
    return _run(inputs, mask.astype(jnp.float32)[:, :, None])


# speedup vs baseline: 1.7657x; 1.7657x over previous
import functools

import jax
import jax.numpy as jnp
from jax import lax
from jax.experimental import pallas as pl
from jax.experimental.pallas import tpu as pltpu
from jax.experimental.pallas import tpu_sc as plsc

B, L, D = 16384, 200, 16
LP = 208
NC, NS = 2, 16
NW = NC * NS
R = B // NW
CR = 8
NCH = R // CR
RD = L * D


def _tree_sum(vs):
    while len(vs) > 1:
        vs = [vs[i] + vs[i + 1] for i in range(0, len(vs) - 1, 2)] + (
            [vs[-1]] if len(vs) % 2 else [])
    return vs[0]


def _body(x_hbm, m_hbm, out_hbm, xbuf0, xbuf1, mbuf, obuf, wbuf, sems):
    xbufs = (xbuf0, xbuf1)
    cid = lax.axis_index("c")
    sid = lax.axis_index("s")
    wid = sid * NC + cid
    base = wid * R

    lane = lax.iota(jnp.int32, 16)
    consts = [lane * 16 + d for d in range(16)]
    zf = jnp.zeros((16,), jnp.float32)

    def start(c, slot):
        row0 = base + c * CR
        pltpu.async_copy(x_hbm.at[pl.ds(row0, CR)],
                         xbufs[slot].at[pl.ds(0, CR)], sems.at[slot])
        pltpu.async_copy(m_hbm.at[pl.ds(row0, CR)], mbuf.at[slot],
                         sems.at[slot])

    def wait(c, slot):
        row0 = base + c * CR
        pltpu.make_async_copy(x_hbm.at[pl.ds(row0, CR)],
                              xbufs[slot].at[pl.ds(0, CR)],
                              sems.at[slot]).wait()
        pltpu.make_async_copy(m_hbm.at[pl.ds(row0, CR)], mbuf.at[slot],
                              sems.at[slot]).wait()

    start(0, 0)
    start(1, 1)

    def process(c, slot):
        wait(c, slot)
        xref = xbufs[slot]
        for r in range(CR):

            def lstep(i, accs):
                mv = mbuf[slot, r, pl.ds(i * 16, 16)]
                mb = mv > 0.0
                idx_l = lane + i * 16
                return tuple(
                    accs[d] + jnp.where(
                        mb,
                        plsc.load_gather(xref, [jnp.full((16,), r, jnp.int32),
                                                idx_l,
                                                jnp.full((16,), d, jnp.int32)]),
                        zf)
                    for d in range(16))

            accs = lax.fori_loop(0, 13, lstep, (zf,) * 16)
            for d in range(16):
                wbuf[pl.ds(d * 16, 16)] = accs[d]
            cols = [plsc.load_gather(wbuf, [consts[u]]) for u in range(16)]
            obuf[r, :] = _tree_sum(cols)
        pltpu.sync_copy(obuf, out_hbm.at[pl.ds(base + c * CR, CR)])

        @pl.when(c + 2 < NCH)
        def _():
            start(c + 2, slot)

    def two_chunks(cp, _):
        process(2 * cp, 0)
        process(2 * cp + 1, 1)
        return 0

    lax.fori_loop(0, NCH // 2, two_chunks, 0)


@jax.jit
def _run(x3d, mpad):
    mesh = plsc.VectorSubcoreMesh(core_axis_name="c", subcore_axis_name="s")
    fn = functools.partial(
        pl.kernel,
        out_type=jax.ShapeDtypeStruct((B, D), jnp.float32),
        mesh=mesh,
        compiler_params=pltpu.CompilerParams(use_tc_tiling_on_sc=False,
                                             needs_layout_passes=False),
        scratch_types=[
            pltpu.VMEM((CR + 1, L, D), jnp.float32),
            pltpu.VMEM((CR + 1, L, D), jnp.float32),
            pltpu.VMEM((2, CR, LP), jnp.float32),
            pltpu.VMEM((CR, D), jnp.float32),
            pltpu.VMEM((256,), jnp.float32),
            pltpu.SemaphoreType.DMA((2,)),
        ],
    )(_body)
    return fn(x3d, mpad)


def kernel(inputs, mask):
    maskf = jnp.pad(mask.astype(jnp.float32), ((0, 0), (0, LP - L)))
    return _run(inputs, maskf)
